# Initial kernel scaffold; baseline (speedup 1.0000x reference)
import jax
import jax.numpy as jnp
from jax.experimental import pallas as pl


def kernel(target, pred):
    raise NotImplementedError("write your pallas kernel here")


# speedup vs baseline: 1.9785x; 1.9785x over previous
import jax
import jax.numpy as jnp
from jax.experimental import pallas as pl

NUM_CLASSES = 1000
SMOOTHING = 0.1
import numpy as np

FILL = float(np.float32(SMOOTHING / NUM_CLASSES))
PEAK = float(np.float32(np.float32(SMOOTHING / NUM_CLASSES) + np.float32(1.0 - SMOOTHING)))

ROWS_PER_BLOCK = 1024


def _tc_body(tgt_ref, out_ref):
    tgt = tgt_ref[0, 0, :].reshape(ROWS_PER_BLOCK, 1)
    cols = jax.lax.broadcasted_iota(jnp.int32, (ROWS_PER_BLOCK, NUM_CLASSES), 1)
    out_ref[...] = jnp.where(cols == tgt, PEAK, FILL)


def kernel(target, pred):
    batch = target.shape[0]
    nblk = batch // ROWS_PER_BLOCK
    tgt3 = target.reshape(nblk, 1, ROWS_PER_BLOCK)
    return pl.pallas_call(
        _tc_body,
        grid=(nblk,),
        in_specs=[pl.BlockSpec((1, 1, ROWS_PER_BLOCK), lambda i: (i, 0, 0))],
        out_specs=pl.BlockSpec((ROWS_PER_BLOCK, NUM_CLASSES), lambda i: (i, 0)),
        out_shape=jax.ShapeDtypeStruct((batch, NUM_CLASSES), jnp.float32),
    )(tgt3)
